# Initial kernel scaffold; baseline (speedup 1.0000x reference)
import jax
import jax.numpy as jnp
from jax.experimental import pallas as pl


def kernel(x, W_box, b_box, W_cls, b_cls, W_iou, W_dir, b_dir, W_var, b_var):
    raise NotImplementedError("write your pallas kernel here")


# speedup vs baseline: 1.0866x; 1.0866x over previous
import jax
import jax.numpy as jnp
from jax.experimental import pallas as pl

_HEAD_DIMS = (14, 2, 4, 14, 2)


def _fused_head_kernel(x_ref, w_ref, b_ref, box_ref, cls_ref, dir_ref,
                       var_ref, iou_ref):
    xt = x_ref[0]
    acc = jax.lax.dot_general(
        xt, w_ref[...], (((0,), (0,)), ((), ())),
        preferred_element_type=jnp.float32)
    acc = acc + b_ref[...]
    box_ref[0] = acc[:, 0:14]
    cls_ref[0] = acc[:, 14:16]
    dir_ref[0] = acc[:, 16:20]
    var_ref[0] = acc[:, 20:34]
    iou_ref[0] = acc[:, 34:36]


def kernel(x, W_box, b_box, W_cls, b_cls, W_iou, W_dir, b_dir, W_var, b_var):
    B, C, H, W = x.shape
    HW = H * W
    Wc = jnp.concatenate([W_box, W_cls, W_dir, W_var, W_iou], axis=0)
    WcT = Wc.T
    bc = jnp.concatenate(
        [b_box, b_cls, b_dir, b_var, jnp.zeros((2,), x.dtype)], axis=0)
    bc2 = bc.reshape(1, 36)

    TH = 16
    MT = TH * W
    nH = H // TH
    x2 = x.reshape(B, C, HW)

    outs = pl.pallas_call(
        _fused_head_kernel,
        grid=(B, nH),
        in_specs=[
            pl.BlockSpec((1, C, MT), lambda b, h: (b, 0, h)),
            pl.BlockSpec((C, 36), lambda b, h: (0, 0)),
            pl.BlockSpec((1, 36), lambda b, h: (0, 0)),
        ],
        out_specs=[
            pl.BlockSpec((1, MT, o), lambda b, h: (b, h, 0))
            for o in _HEAD_DIMS
        ],
        out_shape=[
            jax.ShapeDtypeStruct((B, HW, o), x.dtype) for o in _HEAD_DIMS
        ],
    )(x2, WcT, bc2)

    return tuple(o.reshape(B, H, W, d) for o, d in zip(outs, _HEAD_DIMS))
